# baseline (speedup 1.0000x reference)
import functools
import math

import numpy as np
import jax
import jax.numpy as jnp
from jax import lax
from jax.experimental import pallas as pl
from jax.experimental.pallas import tpu as pltpu
from jax.experimental.pallas import tpu_sc as plsc

N_ITEMS = 1000000
D_DIM = 32
S_SAMPLES = 1024

GROUPS = 4
N_ROWS = N_ITEMS // GROUPS
ROW_BLOCK = 2048
NUM_BLOCKS = N_ROWS // ROW_BLOCK
MAIN_ROWS = NUM_BLOCKS * ROW_BLOCK
TAIL_ROWS = N_ROWS - MAIN_ROWS

NUM_WORKERS = 32
IDX_PER_WORKER = S_SAMPLES // NUM_WORKERS
LANES = 16

_LOG_EPS = math.log(1e-8)

_SEL = (np.arange(128)[:, None] // 32 == np.arange(8)[None, :] % 4)
_SEL = _SEL.astype(np.float32)


def _merge(m_old, z_old, m_b, z_b):
    m_new = jnp.maximum(m_old, m_b)
    z_new = z_old * jnp.exp(m_old - m_new) + z_b * jnp.exp(m_b - m_new)
    return m_new, z_new


def _scores_of(e4, w, bg):
    mm = lax.dot_general(e4, w, (((1,), (0,)), ((), ())),
                         preferred_element_type=jnp.float32)
    st = jnp.transpose(mm)
    return st[0:GROUPS, :] + bg


def _tc_score_body(e_ref, bg_ref, et_ref, bgt_ref, w_ref,
                   s0_ref, s1_ref, s2_ref, s3_ref,
                   t0_ref, t1_ref, t2_ref, t3_ref, c_ref, m_ref, z_ref):
    i = pl.program_id(0)

    @pl.when(i == 0)
    def _():
        m_ref[...] = jnp.full((1, 1), -1e30, jnp.float32)
        z_ref[...] = jnp.zeros((1, 1), jnp.float32)
        c_ref[...] = jnp.zeros((1, 1), jnp.float32)

    w = w_ref[...]
    s4 = _scores_of(e_ref[...], w, bg_ref[...])
    s0_ref[...] = s4[0]
    s1_ref[...] = s4[1]
    s2_ref[...] = s4[2]
    s3_ref[...] = s4[3]
    m_blk = jnp.max(s4)
    z_blk = jnp.sum(jnp.exp(s4 - m_blk))
    m_new, z_new = _merge(m_ref[...], z_ref[...],
                          jnp.full((1, 1), m_blk), jnp.full((1, 1), z_blk))
    m_ref[...] = m_new
    z_ref[...] = z_new

    @pl.when(i == NUM_BLOCKS - 1)
    def _():
        s4t = _scores_of(et_ref[...], w, bgt_ref[...])
        t0_ref[...] = s4t[0]
        t1_ref[...] = s4t[1]
        t2_ref[...] = s4t[2]
        t3_ref[...] = s4t[3]
        mt = jnp.max(s4t)
        zt = jnp.sum(jnp.exp(s4t - mt))
        m_f, z_f = _merge(m_new, z_new,
                          jnp.full((1, 1), mt), jnp.full((1, 1), zt))
        c_ref[...] = m_f + jnp.log(z_f)


def _tc_scores(E4, bg4, e_tail, bg_tail, w):
    main_spec = lambda: pl.BlockSpec((ROW_BLOCK,), lambda i: (i,))
    tail_spec = lambda: pl.BlockSpec((TAIL_ROWS,), lambda i: (0,))
    return pl.pallas_call(
        _tc_score_body,
        grid=(NUM_BLOCKS,),
        in_specs=[
            pl.BlockSpec((ROW_BLOCK, 128), lambda i: (i, 0)),
            pl.BlockSpec((GROUPS, ROW_BLOCK), lambda i: (0, i)),
            pl.BlockSpec((TAIL_ROWS, 128), lambda i: (0, 0)),
            pl.BlockSpec((GROUPS, TAIL_ROWS), lambda i: (0, 0)),
            pl.BlockSpec((128, 8), lambda i: (0, 0)),
        ],
        out_specs=[main_spec(), main_spec(), main_spec(), main_spec(),
                   tail_spec(), tail_spec(), tail_spec(), tail_spec(),
                   pl.BlockSpec((1, 1), lambda i: (0, 0))],
        out_shape=[jax.ShapeDtypeStruct((MAIN_ROWS,), jnp.float32)] * 4
        + [jax.ShapeDtypeStruct((TAIL_ROWS,), jnp.float32)] * 4
        + [jax.ShapeDtypeStruct((1, 1), jnp.float32)],
        scratch_shapes=[
            pltpu.VMEM((1, 1), jnp.float32),
            pltpu.VMEM((1, 1), jnp.float32),
        ],
        compiler_params=pltpu.CompilerParams(
            dimension_semantics=("arbitrary",),
        ),
    )(E4, bg4, e_tail, bg_tail, w)


def _sc_gather(smain, stail, idx):
    mesh = plsc.VectorSubcoreMesh(core_axis_name="c", subcore_axis_name="s")

    @functools.partial(
        pl.kernel,
        mesh=mesh,
        out_type=jax.ShapeDtypeStruct((S_SAMPLES,), jnp.float32),
        scratch_types=[
            pltpu.VMEM((IDX_PER_WORKER,), jnp.int32),
            pltpu.VMEM((IDX_PER_WORKER,), jnp.int32),
            pltpu.VMEM((IDX_PER_WORKER,), jnp.int32),
        ] + [pltpu.VMEM((IDX_PER_WORKER,), jnp.float32) for _ in range(9)]
        + [pltpu.SemaphoreType.DMA],
    )
    def gather_kernel(s0, s1, s2, s3, t0, t1, t2, t3, idx_hbm, out_hbm,
                      idx_v, qm_v, qt_v,
                      gm0, gm1, gm2, gm3, gt0, gt1, gt2, gt3, out_v, sem):
        wid = lax.axis_index("s") * 2 + lax.axis_index("c")
        base = wid * IDX_PER_WORKER
        pltpu.sync_copy(idx_hbm.at[pl.ds(base, IDX_PER_WORKER)], idx_v)
        for c in range(IDX_PER_WORKER // LANES):
            sl = pl.ds(c * LANES, LANES)
            q = lax.shift_right_logical(idx_v[sl], 2)
            qm_v[sl] = jnp.minimum(q, MAIN_ROWS - 1)
            qt_v[sl] = jnp.clip(q - MAIN_ROWS, 0, TAIL_ROWS - 1)
        gms = (gm0, gm1, gm2, gm3)
        gts = (gt0, gt1, gt2, gt3)
        copies = []
        for k, ref in enumerate((s0, s1, s2, s3)):
            copies.append(pltpu.async_copy(ref.at[qm_v], gms[k], sem))
        for k, ref in enumerate((t0, t1, t2, t3)):
            copies.append(pltpu.async_copy(ref.at[qt_v], gts[k], sem))
        for cp in copies:
            cp.wait()
        for c in range(IDX_PER_WORKER // LANES):
            sl = pl.ds(c * LANES, LANES)
            ix = idx_v[sl]
            q = lax.shift_right_logical(ix, 2)
            r = lax.bitwise_and(ix, 3)
            is_main = q < MAIN_ROWS
            vm = gms[0][sl]
            vt = gts[0][sl]
            for k in (1, 2, 3):
                vm = jnp.where(r == k, gms[k][sl], vm)
                vt = jnp.where(r == k, gts[k][sl], vt)
            out_v[sl] = jnp.where(is_main, vm, vt)
        pltpu.sync_copy(out_v, out_hbm.at[pl.ds(base, IDX_PER_WORKER)])

    return gather_kernel(*smain, *stail, idx)


def _tc_loss_body(s_ref, rew_ref, c_ref, out_ref):
    c = c_ref[...].reshape(())
    logp = jnp.maximum(s_ref[...] - c, _LOG_EPS)
    out_ref[...] = jnp.full((1, 1), -jnp.mean(logp * rew_ref[...]))


def kernel(G_user_embeddings, G_item_embeddings, G_item_bias, user_index,
           sample, reward):
    u = lax.dynamic_slice_in_dim(
        G_user_embeddings, user_index, 1, axis=0).reshape((D_DIM,))
    idx = sample.astype(jnp.int32)

    E4 = G_item_embeddings.reshape(N_ROWS, GROUPS * D_DIM)
    w = jnp.tile(u, GROUPS)[:, None] * jnp.asarray(_SEL)
    bg4 = jnp.transpose(G_item_bias.reshape(N_ROWS, GROUPS))

    e_tail = lax.slice(E4, (MAIN_ROWS, 0), (N_ROWS, GROUPS * D_DIM))
    bg_tail = lax.slice(bg4, (0, MAIN_ROWS), (GROUPS, N_ROWS))
    bg_main = lax.slice(bg4, (0, 0), (GROUPS, MAIN_ROWS))

    outs = _tc_scores(E4, bg_main, e_tail, bg_tail, w)
    smain, stail, c = outs[0:4], outs[4:8], outs[8]

    s_smp = _sc_gather(smain, stail, idx)

    loss = pl.pallas_call(
        _tc_loss_body,
        out_shape=jax.ShapeDtypeStruct((1, 1), jnp.float32),
    )(s_smp, reward, c)
    return loss.reshape(())


# speedup vs baseline: 5.2692x; 5.2692x over previous
import functools
import math

import jax
import jax.numpy as jnp
from jax import lax
from jax.experimental import pallas as pl
from jax.experimental.pallas import tpu as pltpu
from jax.experimental.pallas import tpu_sc as plsc

N_ITEMS = 1000000
D_DIM = 32
S_SAMPLES = 1024

BLOCK_ITEMS = 8192
NUM_BLOCKS = N_ITEMS // BLOCK_ITEMS
MAIN_ITEMS = NUM_BLOCKS * BLOCK_ITEMS
TAIL_ITEMS = N_ITEMS - MAIN_ITEMS

NUM_WORKERS = 32
IDX_PER_WORKER = S_SAMPLES // NUM_WORKERS
LANES = 16

_LOG_EPS = math.log(1e-8)


def _merge(m_old, z_old, m_b, z_b):
    m_new = jnp.maximum(m_old, m_b)
    z_new = z_old * jnp.exp(m_old - m_new) + z_b * jnp.exp(m_b - m_new)
    return m_new, z_new


def _tc_score_body(et_ref, b_ref, ett_ref, bt_ref, ut_ref,
                   smain_ref, stail_ref, c_ref, m_ref, z_ref):
    i = pl.program_id(0)

    @pl.when(i == 0)
    def _():
        m_ref[...] = jnp.full((1, 1), -1e30, jnp.float32)
        z_ref[...] = jnp.zeros((1, 1), jnp.float32)
        c_ref[...] = jnp.zeros((1, 1), jnp.float32)
        stail_ref[...] = jnp.zeros((TAIL_ITEMS,), jnp.float32)

    ut = ut_ref[...]
    s = jnp.sum(et_ref[...] * ut, axis=0) + b_ref[...]
    smain_ref[...] = s
    m_blk = jnp.max(s)
    z_blk = jnp.sum(jnp.exp(s - m_blk))
    m_new, z_new = _merge(m_ref[...], z_ref[...],
                          jnp.full((1, 1), m_blk), jnp.full((1, 1), z_blk))
    m_ref[...] = m_new
    z_ref[...] = z_new

    @pl.when(i == NUM_BLOCKS - 1)
    def _():
        st = jnp.sum(ett_ref[...] * ut, axis=0) + bt_ref[...]
        stail_ref[...] = st
        mt = jnp.max(st)
        zt = jnp.sum(jnp.exp(st - mt))
        m_f, z_f = _merge(m_new, z_new,
                          jnp.full((1, 1), mt), jnp.full((1, 1), zt))
        c_ref[...] = m_f + jnp.log(z_f)


def _tc_scores(Et, B, et_tail, b_tail, ut):
    return pl.pallas_call(
        _tc_score_body,
        grid=(NUM_BLOCKS,),
        in_specs=[
            pl.BlockSpec((D_DIM, BLOCK_ITEMS), lambda i: (0, i)),
            pl.BlockSpec((BLOCK_ITEMS,), lambda i: (i,)),
            pl.BlockSpec((D_DIM, TAIL_ITEMS), lambda i: (0, 0)),
            pl.BlockSpec((TAIL_ITEMS,), lambda i: (0,)),
            pl.BlockSpec((D_DIM, 1), lambda i: (0, 0)),
        ],
        out_specs=[
            pl.BlockSpec((BLOCK_ITEMS,), lambda i: (i,)),
            pl.BlockSpec((TAIL_ITEMS,), lambda i: (0,)),
            pl.BlockSpec((1, 1), lambda i: (0, 0)),
        ],
        out_shape=[
            jax.ShapeDtypeStruct((MAIN_ITEMS,), jnp.float32),
            jax.ShapeDtypeStruct((TAIL_ITEMS,), jnp.float32),
            jax.ShapeDtypeStruct((1, 1), jnp.float32),
        ],
        scratch_shapes=[
            pltpu.VMEM((1, 1), jnp.float32),
            pltpu.VMEM((1, 1), jnp.float32),
        ],
        compiler_params=pltpu.CompilerParams(
            dimension_semantics=("arbitrary",),
        ),
    )(Et, B, et_tail, b_tail, ut)


def _sc_gather(s_main, s_tail, idx):
    mesh = plsc.VectorSubcoreMesh(core_axis_name="c", subcore_axis_name="s")

    @functools.partial(
        pl.kernel,
        mesh=mesh,
        out_type=jax.ShapeDtypeStruct((S_SAMPLES,), jnp.float32),
        scratch_types=[
            pltpu.VMEM((IDX_PER_WORKER,), jnp.int32),
            pltpu.VMEM((IDX_PER_WORKER,), jnp.int32),
            pltpu.VMEM((IDX_PER_WORKER,), jnp.int32),
            pltpu.VMEM((IDX_PER_WORKER,), jnp.float32),
            pltpu.VMEM((IDX_PER_WORKER,), jnp.float32),
            pltpu.VMEM((IDX_PER_WORKER,), jnp.float32),
            pltpu.SemaphoreType.DMA,
            pltpu.SemaphoreType.DMA,
        ],
    )
    def gather_kernel(smain_hbm, stail_hbm, idx_hbm, out_hbm,
                      idx_v, im_v, it_v, gm_v, gt_v, out_v, sem_m, sem_t):
        wid = lax.axis_index("s") * 2 + lax.axis_index("c")
        base = wid * IDX_PER_WORKER
        pltpu.sync_copy(idx_hbm.at[pl.ds(base, IDX_PER_WORKER)], idx_v)
        for c in range(IDX_PER_WORKER // LANES):
            sl = pl.ds(c * LANES, LANES)
            ix = idx_v[sl]
            im_v[sl] = jnp.minimum(ix, MAIN_ITEMS - 1)
            it_v[sl] = jnp.clip(ix - MAIN_ITEMS, 0, TAIL_ITEMS - 1)
        cp_m = pltpu.async_copy(smain_hbm.at[im_v], gm_v, sem_m)
        cp_t = pltpu.async_copy(stail_hbm.at[it_v], gt_v, sem_t)
        cp_m.wait()
        cp_t.wait()
        for c in range(IDX_PER_WORKER // LANES):
            sl = pl.ds(c * LANES, LANES)
            out_v[sl] = jnp.where(idx_v[sl] < MAIN_ITEMS, gm_v[sl], gt_v[sl])
        pltpu.sync_copy(out_v, out_hbm.at[pl.ds(base, IDX_PER_WORKER)])

    return gather_kernel(s_main, s_tail, idx)


def _tc_loss_body(s_ref, rew_ref, c_ref, out_ref):
    c = c_ref[...].reshape(())
    logp = jnp.maximum(s_ref[...] - c, _LOG_EPS)
    out_ref[...] = jnp.full((1, 1), -jnp.mean(logp * rew_ref[...]))


def kernel(G_user_embeddings, G_item_embeddings, G_item_bias, user_index,
           sample, reward):
    ut = jnp.transpose(
        lax.dynamic_slice_in_dim(G_user_embeddings, user_index, 1, axis=0))
    idx = sample.astype(jnp.int32)

    Et = jnp.transpose(G_item_embeddings)
    et_tail = lax.slice(Et, (0, MAIN_ITEMS), (D_DIM, N_ITEMS))
    b_tail = lax.slice(G_item_bias, (MAIN_ITEMS,), (N_ITEMS,))

    s_main, s_tail, c = _tc_scores(Et, G_item_bias, et_tail, b_tail, ut)
    s_smp = _sc_gather(s_main, s_tail, idx)

    loss = pl.pallas_call(
        _tc_loss_body,
        out_shape=jax.ShapeDtypeStruct((1, 1), jnp.float32),
    )(s_smp, reward, c)
    return loss.reshape(())


# speedup vs baseline: 7.8479x; 1.4894x over previous
import functools
import math

import jax
import jax.numpy as jnp
from jax import lax
from jax.experimental import pallas as pl
from jax.experimental.pallas import tpu as pltpu
from jax.experimental.pallas import tpu_sc as plsc

N_ITEMS = 1000000
D_DIM = 32
S_SAMPLES = 1024

BLOCK_ITEMS = 16384
NUM_BLOCKS = -(-N_ITEMS // BLOCK_ITEMS)

NUM_WORKERS = 32
IDX_PER_WORKER = S_SAMPLES // NUM_WORKERS
LANES = 16

_LOG_EPS = math.log(1e-8)
_NEG_BIG = -1e30


def _tc_score_body(et_ref, b_ref, ut_ref, s_ref, m_ref, z_ref):
    i = pl.program_id(0)
    s = lax.dot_general(ut_ref[...], et_ref[...], (((1,), (0,)), ((), ())),
                        preferred_element_type=jnp.float32
                        ).reshape((BLOCK_ITEMS,)) + b_ref[...]
    s_ref[...] = s
    pos = i * BLOCK_ITEMS + lax.iota(jnp.int32, BLOCK_ITEMS)
    sm = jnp.where(pos < N_ITEMS, s, _NEG_BIG)
    m_blk = jnp.max(sm)
    z_blk = jnp.sum(jnp.exp(sm - m_blk))
    m_ref[...] = jnp.full((128,), m_blk)
    z_ref[...] = jnp.full((128,), z_blk)


def _tc_scores(Et, B, ut):
    return pl.pallas_call(
        _tc_score_body,
        grid=(NUM_BLOCKS,),
        in_specs=[
            pl.BlockSpec((D_DIM, BLOCK_ITEMS), lambda i: (0, i)),
            pl.BlockSpec((BLOCK_ITEMS,), lambda i: (i,)),
            pl.BlockSpec((1, D_DIM), lambda i: (0, 0)),
        ],
        out_specs=[
            pl.BlockSpec((BLOCK_ITEMS,), lambda i: (i,)),
            pl.BlockSpec((128,), lambda i: (i,)),
            pl.BlockSpec((128,), lambda i: (i,)),
        ],
        out_shape=[
            jax.ShapeDtypeStruct((N_ITEMS,), jnp.float32),
            jax.ShapeDtypeStruct((NUM_BLOCKS * 128,), jnp.float32),
            jax.ShapeDtypeStruct((NUM_BLOCKS * 128,), jnp.float32),
        ],
        compiler_params=pltpu.CompilerParams(
            dimension_semantics=("parallel",),
        ),
    )(Et, B, ut)


def _sc_gather(scores, idx):
    mesh = plsc.VectorSubcoreMesh(core_axis_name="c", subcore_axis_name="s")

    @functools.partial(
        pl.kernel,
        mesh=mesh,
        out_type=jax.ShapeDtypeStruct((S_SAMPLES,), jnp.float32),
        scratch_types=[
            pltpu.VMEM((IDX_PER_WORKER,), jnp.int32),
            pltpu.VMEM((IDX_PER_WORKER,), jnp.float32),
            pltpu.SemaphoreType.DMA,
        ],
    )
    def gather_kernel(s_hbm, idx_hbm, out_hbm, idx_v, g_v, sem):
        wid = lax.axis_index("s") * 2 + lax.axis_index("c")
        base = wid * IDX_PER_WORKER
        pltpu.sync_copy(idx_hbm.at[pl.ds(base, IDX_PER_WORKER)], idx_v)
        cp = pltpu.async_copy(s_hbm.at[idx_v], g_v, sem)
        cp.wait()
        pltpu.sync_copy(g_v, out_hbm.at[pl.ds(base, IDX_PER_WORKER)])

    return gather_kernel(scores, idx)


def _tc_loss_body(s_ref, rew_ref, m_ref, z_ref, out_ref):
    m_vec = m_ref[...]
    z_vec = z_ref[...]
    m_all = jnp.max(m_vec)
    z_all = jnp.sum(z_vec * jnp.exp(m_vec - m_all)) * (1.0 / 128.0)
    c = m_all + jnp.log(z_all)
    logp = jnp.maximum(s_ref[...] - c, _LOG_EPS)
    out_ref[...] = jnp.full((1, 1), -jnp.mean(logp * rew_ref[...]))


def kernel(G_user_embeddings, G_item_embeddings, G_item_bias, user_index,
           sample, reward):
    ut = lax.dynamic_slice_in_dim(G_user_embeddings, user_index, 1, axis=0)
    idx = sample.astype(jnp.int32)

    Et = jnp.transpose(G_item_embeddings)

    s, m_vec, z_vec = _tc_scores(Et, G_item_bias, ut)
    s_smp = _sc_gather(s, idx)

    loss = pl.pallas_call(
        _tc_loss_body,
        out_shape=jax.ShapeDtypeStruct((1, 1), jnp.float32),
    )(s_smp, reward, m_vec, z_vec)
    return loss.reshape(())


# speedup vs baseline: 10.0456x; 1.2800x over previous
import functools
import math

import jax
import jax.numpy as jnp
from jax import lax
from jax.experimental import pallas as pl
from jax.experimental.pallas import tpu as pltpu
from jax.experimental.pallas import tpu_sc as plsc

N_ITEMS = 1000000
D_DIM = 32
S_SAMPLES = 1024

BLOCK_ITEMS = 32768
NUM_BLOCKS = -(-N_ITEMS // BLOCK_ITEMS)

NUM_WORKERS = 32
IDX_PER_WORKER = S_SAMPLES // NUM_WORKERS
LANES = 16

_LOG_EPS = math.log(1e-8)
_NEG_BIG = -1e30


def _tc_score_body(et_ref, b_ref, ut_ref, s_ref, m_ref, z_ref):
    i = pl.program_id(0)
    s = lax.dot_general(ut_ref[...], et_ref[...], (((1,), (0,)), ((), ())),
                        preferred_element_type=jnp.float32
                        ).reshape((BLOCK_ITEMS,)) + b_ref[...]
    s_ref[...] = s
    pos = i * BLOCK_ITEMS + lax.iota(jnp.int32, BLOCK_ITEMS)
    sm = jnp.where(pos < N_ITEMS, s, _NEG_BIG)
    m_blk = jnp.max(sm)
    z_blk = jnp.sum(jnp.exp(sm - m_blk))
    m_ref[...] = jnp.full((128,), m_blk)
    z_ref[...] = jnp.full((128,), z_blk)


def _tc_scores(Et, B, ut):
    return pl.pallas_call(
        _tc_score_body,
        grid=(NUM_BLOCKS,),
        in_specs=[
            pl.BlockSpec((D_DIM, BLOCK_ITEMS), lambda i: (0, i)),
            pl.BlockSpec((BLOCK_ITEMS,), lambda i: (i,)),
            pl.BlockSpec((1, D_DIM), lambda i: (0, 0)),
        ],
        out_specs=[
            pl.BlockSpec((BLOCK_ITEMS,), lambda i: (i,)),
            pl.BlockSpec((128,), lambda i: (i,)),
            pl.BlockSpec((128,), lambda i: (i,)),
        ],
        out_shape=[
            jax.ShapeDtypeStruct((N_ITEMS,), jnp.float32),
            jax.ShapeDtypeStruct((NUM_BLOCKS * 128,), jnp.float32),
            jax.ShapeDtypeStruct((NUM_BLOCKS * 128,), jnp.float32),
        ],
        compiler_params=pltpu.CompilerParams(
            dimension_semantics=("parallel",),
        ),
    )(Et, B, ut)


def _sc_gather(scores, idx):
    mesh = plsc.VectorSubcoreMesh(core_axis_name="c", subcore_axis_name="s")

    @functools.partial(
        pl.kernel,
        mesh=mesh,
        out_type=jax.ShapeDtypeStruct((S_SAMPLES,), jnp.float32),
        scratch_types=[
            pltpu.VMEM((IDX_PER_WORKER,), jnp.int32),
            pltpu.VMEM((IDX_PER_WORKER,), jnp.float32),
            pltpu.SemaphoreType.DMA,
        ],
    )
    def gather_kernel(s_hbm, idx_hbm, out_hbm, idx_v, g_v, sem):
        wid = lax.axis_index("s") * 2 + lax.axis_index("c")
        base = wid * IDX_PER_WORKER
        pltpu.sync_copy(idx_hbm.at[pl.ds(base, IDX_PER_WORKER)], idx_v)
        cp = pltpu.async_copy(s_hbm.at[idx_v], g_v, sem)
        cp.wait()
        pltpu.sync_copy(g_v, out_hbm.at[pl.ds(base, IDX_PER_WORKER)])

    return gather_kernel(scores, idx)


def _tc_loss_body(s_ref, rew_ref, m_ref, z_ref, out_ref):
    m_vec = m_ref[...]
    z_vec = z_ref[...]
    m_all = jnp.max(m_vec)
    z_all = jnp.sum(z_vec * jnp.exp(m_vec - m_all)) * (1.0 / 128.0)
    c = m_all + jnp.log(z_all)
    logp = jnp.maximum(s_ref[...] - c, _LOG_EPS)
    out_ref[...] = jnp.full((1, 1), -jnp.mean(logp * rew_ref[...]))


def kernel(G_user_embeddings, G_item_embeddings, G_item_bias, user_index,
           sample, reward):
    ut = lax.dynamic_slice_in_dim(G_user_embeddings, user_index, 1, axis=0)
    idx = sample.astype(jnp.int32)

    Et = jnp.transpose(G_item_embeddings)

    s, m_vec, z_vec = _tc_scores(Et, G_item_bias, ut)
    s_smp = _sc_gather(s, idx)

    loss = pl.pallas_call(
        _tc_loss_body,
        out_shape=jax.ShapeDtypeStruct((1, 1), jnp.float32),
    )(s_smp, reward, m_vec, z_vec)
    return loss.reshape(())


# speedup vs baseline: 11.4321x; 1.1380x over previous
import functools
import math

import jax
import jax.numpy as jnp
from jax import lax
from jax.experimental import pallas as pl
from jax.experimental.pallas import tpu as pltpu
from jax.experimental.pallas import tpu_sc as plsc

N_ITEMS = 1000000
D_DIM = 32
S_SAMPLES = 1024

BLOCK_ITEMS = 65536
NUM_BLOCKS = -(-N_ITEMS // BLOCK_ITEMS)

NUM_WORKERS = 32
IDX_PER_WORKER = S_SAMPLES // NUM_WORKERS
LANES = 16

_LOG_EPS = math.log(1e-8)
_NEG_BIG = -1e30


def _tc_score_body(et_ref, b_ref, ut_ref, s_ref, m_ref, z_ref):
    i = pl.program_id(0)
    s = lax.dot_general(ut_ref[...], et_ref[...], (((1,), (0,)), ((), ())),
                        preferred_element_type=jnp.float32
                        ).reshape((BLOCK_ITEMS,)) + b_ref[...]
    s_ref[...] = s
    pos = i * BLOCK_ITEMS + lax.iota(jnp.int32, BLOCK_ITEMS)
    sm = jnp.where(pos < N_ITEMS, s, _NEG_BIG)
    m_blk = jnp.max(sm)
    z_blk = jnp.sum(jnp.exp(sm - m_blk))
    m_ref[...] = jnp.full((128,), m_blk)
    z_ref[...] = jnp.full((128,), z_blk)


def _tc_scores(Et, B, ut):
    return pl.pallas_call(
        _tc_score_body,
        grid=(NUM_BLOCKS,),
        in_specs=[
            pl.BlockSpec((D_DIM, BLOCK_ITEMS), lambda i: (0, i)),
            pl.BlockSpec((BLOCK_ITEMS,), lambda i: (i,)),
            pl.BlockSpec((1, D_DIM), lambda i: (0, 0)),
        ],
        out_specs=[
            pl.BlockSpec((BLOCK_ITEMS,), lambda i: (i,)),
            pl.BlockSpec((128,), lambda i: (i,)),
            pl.BlockSpec((128,), lambda i: (i,)),
        ],
        out_shape=[
            jax.ShapeDtypeStruct((N_ITEMS,), jnp.float32),
            jax.ShapeDtypeStruct((NUM_BLOCKS * 128,), jnp.float32),
            jax.ShapeDtypeStruct((NUM_BLOCKS * 128,), jnp.float32),
        ],
        compiler_params=pltpu.CompilerParams(
            dimension_semantics=("parallel",),
        ),
    )(Et, B, ut)


def _sc_gather(scores, idx):
    mesh = plsc.VectorSubcoreMesh(core_axis_name="c", subcore_axis_name="s")

    @functools.partial(
        pl.kernel,
        mesh=mesh,
        out_type=jax.ShapeDtypeStruct((S_SAMPLES,), jnp.float32),
        scratch_types=[
            pltpu.VMEM((IDX_PER_WORKER,), jnp.int32),
            pltpu.VMEM((IDX_PER_WORKER,), jnp.float32),
            pltpu.SemaphoreType.DMA,
        ],
    )
    def gather_kernel(s_hbm, idx_hbm, out_hbm, idx_v, g_v, sem):
        wid = lax.axis_index("s") * 2 + lax.axis_index("c")
        base = wid * IDX_PER_WORKER
        pltpu.sync_copy(idx_hbm.at[pl.ds(base, IDX_PER_WORKER)], idx_v)
        cp = pltpu.async_copy(s_hbm.at[idx_v], g_v, sem)
        cp.wait()
        pltpu.sync_copy(g_v, out_hbm.at[pl.ds(base, IDX_PER_WORKER)])

    return gather_kernel(scores, idx)


def _tc_loss_body(s_ref, rew_ref, m_ref, z_ref, out_ref):
    m_vec = m_ref[...]
    z_vec = z_ref[...]
    m_all = jnp.max(m_vec)
    z_all = jnp.sum(z_vec * jnp.exp(m_vec - m_all)) * (1.0 / 128.0)
    c = m_all + jnp.log(z_all)
    logp = jnp.maximum(s_ref[...] - c, _LOG_EPS)
    out_ref[...] = jnp.full((1, 1), -jnp.mean(logp * rew_ref[...]))


def kernel(G_user_embeddings, G_item_embeddings, G_item_bias, user_index,
           sample, reward):
    ut = lax.dynamic_slice_in_dim(G_user_embeddings, user_index, 1, axis=0)
    idx = sample.astype(jnp.int32)

    Et = jnp.transpose(G_item_embeddings)

    s, m_vec, z_vec = _tc_scores(Et, G_item_bias, ut)
    s_smp = _sc_gather(s, idx)

    loss = pl.pallas_call(
        _tc_loss_body,
        out_shape=jax.ShapeDtypeStruct((1, 1), jnp.float32),
    )(s_smp, reward, m_vec, z_vec)
    return loss.reshape(())


# speedup vs baseline: 11.6340x; 1.0177x over previous
import functools
import math

import jax
import jax.numpy as jnp
from jax import lax
from jax.experimental import pallas as pl
from jax.experimental.pallas import tpu as pltpu
from jax.experimental.pallas import tpu_sc as plsc

N_ITEMS = 1000000
D_DIM = 32
S_SAMPLES = 1024

BLOCK_ITEMS = 131072
NUM_BLOCKS = -(-N_ITEMS // BLOCK_ITEMS)

NUM_WORKERS = 32
IDX_PER_WORKER = S_SAMPLES // NUM_WORKERS
LANES = 16

_LOG_EPS = math.log(1e-8)
_NEG_BIG = -1e30


def _tc_score_body(et_ref, b_ref, ut_ref, s_ref, m_ref, z_ref):
    i = pl.program_id(0)
    s = lax.dot_general(ut_ref[...], et_ref[...], (((1,), (0,)), ((), ())),
                        preferred_element_type=jnp.float32
                        ).reshape((BLOCK_ITEMS,)) + b_ref[...]
    s_ref[...] = s
    pos = i * BLOCK_ITEMS + lax.iota(jnp.int32, BLOCK_ITEMS)
    sm = jnp.where(pos < N_ITEMS, s, _NEG_BIG)
    m_blk = jnp.max(sm)
    z_blk = jnp.sum(jnp.exp(sm - m_blk))
    m_ref[...] = jnp.full((128,), m_blk)
    z_ref[...] = jnp.full((128,), z_blk)


def _tc_scores(Et, B, ut):
    return pl.pallas_call(
        _tc_score_body,
        grid=(NUM_BLOCKS,),
        in_specs=[
            pl.BlockSpec((D_DIM, BLOCK_ITEMS), lambda i: (0, i)),
            pl.BlockSpec((BLOCK_ITEMS,), lambda i: (i,)),
            pl.BlockSpec((1, D_DIM), lambda i: (0, 0)),
        ],
        out_specs=[
            pl.BlockSpec((BLOCK_ITEMS,), lambda i: (i,)),
            pl.BlockSpec((128,), lambda i: (i,)),
            pl.BlockSpec((128,), lambda i: (i,)),
        ],
        out_shape=[
            jax.ShapeDtypeStruct((N_ITEMS,), jnp.float32),
            jax.ShapeDtypeStruct((NUM_BLOCKS * 128,), jnp.float32),
            jax.ShapeDtypeStruct((NUM_BLOCKS * 128,), jnp.float32),
        ],
        compiler_params=pltpu.CompilerParams(
            dimension_semantics=("parallel",),
        ),
    )(Et, B, ut)


def _sc_gather(scores, idx):
    mesh = plsc.VectorSubcoreMesh(core_axis_name="c", subcore_axis_name="s")

    @functools.partial(
        pl.kernel,
        mesh=mesh,
        out_type=jax.ShapeDtypeStruct((S_SAMPLES,), jnp.float32),
        scratch_types=[
            pltpu.VMEM((IDX_PER_WORKER,), jnp.int32),
            pltpu.VMEM((IDX_PER_WORKER,), jnp.float32),
            pltpu.SemaphoreType.DMA,
        ],
    )
    def gather_kernel(s_hbm, idx_hbm, out_hbm, idx_v, g_v, sem):
        wid = lax.axis_index("s") * 2 + lax.axis_index("c")
        base = wid * IDX_PER_WORKER
        pltpu.sync_copy(idx_hbm.at[pl.ds(base, IDX_PER_WORKER)], idx_v)
        cp = pltpu.async_copy(s_hbm.at[idx_v], g_v, sem)
        cp.wait()
        pltpu.sync_copy(g_v, out_hbm.at[pl.ds(base, IDX_PER_WORKER)])

    return gather_kernel(scores, idx)


def _tc_loss_body(s_ref, rew_ref, m_ref, z_ref, out_ref):
    m_vec = m_ref[...]
    z_vec = z_ref[...]
    m_all = jnp.max(m_vec)
    z_all = jnp.sum(z_vec * jnp.exp(m_vec - m_all)) * (1.0 / 128.0)
    c = m_all + jnp.log(z_all)
    logp = jnp.maximum(s_ref[...] - c, _LOG_EPS)
    out_ref[...] = jnp.full((1, 1), -jnp.mean(logp * rew_ref[...]))


def kernel(G_user_embeddings, G_item_embeddings, G_item_bias, user_index,
           sample, reward):
    ut = lax.dynamic_slice_in_dim(G_user_embeddings, user_index, 1, axis=0)
    idx = sample.astype(jnp.int32)

    Et = jnp.transpose(G_item_embeddings)

    s, m_vec, z_vec = _tc_scores(Et, G_item_bias, ut)
    s_smp = _sc_gather(s, idx)

    loss = pl.pallas_call(
        _tc_loss_body,
        out_shape=jax.ShapeDtypeStruct((1, 1), jnp.float32),
    )(s_smp, reward, m_vec, z_vec)
    return loss.reshape(())
